# baseline (speedup 1.0000x reference)
import functools

import jax
import jax.numpy as jnp
from jax import lax
from jax.experimental import pallas as pl
from jax.experimental.pallas import tpu as pltpu
from jax.experimental.pallas import tpu_sc as plsc

_N = 375
_TOTAL = _N * _N
_NW = 32
_CHUNK = 4400
_LAST_BASE = (_NW - 1) * _CHUNK
_LAST = _TOTAL - _LAST_BASE
_TAIL_FILL = _LAST - 1


def _make_sc_gather():
    mesh = plsc.VectorSubcoreMesh(core_axis_name="c", subcore_axis_name="s")

    @functools.partial(
        pl.kernel,
        mesh=mesh,
        out_type=jax.ShapeDtypeStruct((_TOTAL,), jnp.float32),
        scratch_types=[
            pltpu.VMEM_SHARED((14,), jnp.float32),
            pltpu.VMEM((_CHUNK,), jnp.int32),
            pltpu.VMEM((_CHUNK,), jnp.float32),
            pltpu.SemaphoreType.DMA,
        ],
    )
    def gather_kernel(idx_hbm, tab_hbm, out_hbm, tab_s, idx_v, val_v, sem):
        cid = lax.axis_index("c")
        sid = lax.axis_index("s")
        wid = sid * 2 + cid
        base = wid * _CHUNK

        @pl.when(sid == 0)
        def _():
            pltpu.sync_copy(tab_hbm, tab_s)

        plsc.subcore_barrier()

        def run(n, zerofill):
            if zerofill:
                zeros = jnp.zeros((16,), jnp.int32)
                for k in range(_TAIL_FILL, _CHUNK, 16):
                    idx_v[pl.ds(k, 16)] = zeros
            pltpu.sync_copy(
                idx_hbm.at[pl.ds(base, n)], idx_v.at[pl.ds(0, n)]
            )
            pltpu.async_copy(tab_s.at[idx_v], val_v, sem).wait()
            pltpu.sync_copy(
                val_v.at[pl.ds(0, n)], out_hbm.at[pl.ds(base, n)]
            )

        @pl.when(wid < _NW - 1)
        def _():
            run(_CHUNK, False)

        @pl.when(wid == _NW - 1)
        def _():
            run(_LAST, True)

    return gather_kernel


_sc_gather = _make_sc_gather()


def kernel(dist_par_0, par_table0):
    flat = dist_par_0.reshape(-1)
    tab = par_table0.reshape(-1)
    out = _sc_gather(flat, tab)
    return out.reshape(_N, _N)


# speedup vs baseline: 1.4491x; 1.4491x over previous
import functools

import jax
import jax.numpy as jnp
from jax import lax
from jax.experimental import pallas as pl
from jax.experimental.pallas import tpu as pltpu
from jax.experimental.pallas import tpu_sc as plsc

_N = 375
_ROWS_PER = 16
_FULL = _N // _ROWS_PER
_TAIL_ROWS = _N - _FULL * _ROWS_PER
_COLS = list(range(0, _N - 16, 16)) + [_N - 16]


def _make_sc_gather():
    mesh = plsc.VectorSubcoreMesh(core_axis_name="c", subcore_axis_name="s")

    @functools.partial(
        pl.kernel,
        mesh=mesh,
        out_type=(
            jax.ShapeDtypeStruct((_N, _N), jnp.float32),
            jax.ShapeDtypeStruct((_TAIL_ROWS, _N), jnp.float32),
        ),
        scratch_types=[
            pltpu.VMEM((16,), jnp.float32),
            pltpu.VMEM((_ROWS_PER, _N), jnp.int32),
            pltpu.VMEM((_ROWS_PER, _N), jnp.float32),
        ],
    )
    def gather_kernel(idx_hbm, idxt_hbm, tab_hbm, out_hbm, outt_hbm,
                      tab_v, idx_v, val_v):
        wid = lax.axis_index("s") * 2 + lax.axis_index("c")

        dnums = lax.GatherDimensionNumbers(
            offset_dims=(), collapsed_slice_dims=(0,), start_index_map=(0,)
        )

        def lookup(nrows):
            tab_vec = tab_v[...]
            for r in range(nrows):
                for c in _COLS:
                    idx16 = idx_v[r, pl.ds(c, 16)]
                    val_v[r, pl.ds(c, 16)] = lax.gather(
                        tab_vec,
                        idx16[:, None],
                        dnums,
                        slice_sizes=(1,),
                        mode=lax.GatherScatterMode.PROMISE_IN_BOUNDS,
                    )

        @pl.when(wid < _FULL)
        def _():
            r0 = wid * _ROWS_PER
            pltpu.sync_copy(tab_hbm, tab_v.at[pl.ds(0, 14)])
            pltpu.sync_copy(idx_hbm.at[pl.ds(r0, _ROWS_PER)], idx_v)
            lookup(_ROWS_PER)
            pltpu.sync_copy(val_v, out_hbm.at[pl.ds(r0, _ROWS_PER)])

        @pl.when(wid == _FULL)
        def _():
            pltpu.sync_copy(tab_hbm, tab_v.at[pl.ds(0, 14)])
            pltpu.sync_copy(idxt_hbm, idx_v.at[pl.ds(0, _TAIL_ROWS)])
            lookup(_TAIL_ROWS)
            pltpu.sync_copy(val_v.at[pl.ds(0, _TAIL_ROWS)], outt_hbm)

    return gather_kernel


_sc_gather = _make_sc_gather()


def kernel(dist_par_0, par_table0):
    tail = lax.slice(dist_par_0, (_FULL * _ROWS_PER, 0), (_N, _N))
    out, out_tail = _sc_gather(dist_par_0, tail, par_table0.reshape(-1))
    return lax.dynamic_update_slice(out, out_tail, (_FULL * _ROWS_PER, 0))


# speedup vs baseline: 1.4962x; 1.0325x over previous
import functools

import jax
import jax.numpy as jnp
from jax import lax
from jax.experimental import pallas as pl
from jax.experimental.pallas import tpu as pltpu
from jax.experimental.pallas import tpu_sc as plsc

_N = 375
_ROWS_PER = 16
_FULL = _N // _ROWS_PER
_TAIL0 = _FULL * _ROWS_PER
_COLS = list(range(0, _N - 16, 16)) + [_N - 16]
_NVALS = 14


def _make_sc_gather():
    mesh = plsc.VectorSubcoreMesh(core_axis_name="c", subcore_axis_name="s")

    @functools.partial(
        pl.kernel,
        mesh=mesh,
        out_type=jax.ShapeDtypeStruct((_N, _N), jnp.float32),
        scratch_types=[
            pltpu.VMEM((16,), jnp.float32),
            pltpu.VMEM((_ROWS_PER, _N), jnp.int32),
            pltpu.VMEM((_ROWS_PER, _N), jnp.float32),
            pltpu.SemaphoreType.DMA,
            pltpu.SemaphoreType.DMA,
        ],
    )
    def gather_kernel(idx_hbm, tab_hbm, out_hbm, tab_v, idx_v, val_v, s0, s1):
        wid = lax.axis_index("s") * 2 + lax.axis_index("c")

        dnums = lax.GatherDimensionNumbers(
            offset_dims=(), collapsed_slice_dims=(0,), start_index_map=(0,)
        )

        @pl.when(wid < _FULL)
        def _():
            r0 = wid * _ROWS_PER
            tab_cp = pltpu.async_copy(tab_hbm, tab_v.at[pl.ds(0, _NVALS)], s0)
            idx_cp = pltpu.async_copy(
                idx_hbm.at[pl.ds(r0, _ROWS_PER)], idx_v, s1
            )
            tab_cp.wait()
            tab_vec = tab_v[...]
            idx_cp.wait()
            for r in range(_ROWS_PER):
                for c in _COLS:
                    idx16 = idx_v[r, pl.ds(c, 16)]
                    val_v[r, pl.ds(c, 16)] = lax.gather(
                        tab_vec,
                        idx16[:, None],
                        dnums,
                        slice_sizes=(1,),
                        mode=lax.GatherScatterMode.PROMISE_IN_BOUNDS,
                    )
            pltpu.sync_copy(val_v, out_hbm.at[pl.ds(r0, _ROWS_PER)])

    return gather_kernel


_sc_gather = _make_sc_gather()


def _tail_body(idx_ref, tab_ref, main_ref, out_ref):
    del main_ref
    x = idx_ref[...]
    acc = jnp.full(x.shape, tab_ref[0, 0], jnp.float32)
    for k in range(1, _NVALS):
        acc = jnp.where(x == k, tab_ref[k, 0], acc)
    out_ref[...] = acc


def _tail_fill(dist, table, main):
    blk = pl.BlockSpec((8, _N), lambda i: (_TAIL0 // 8, 0))
    return pl.pallas_call(
        _tail_body,
        grid=(1,),
        in_specs=[
            blk,
            pl.BlockSpec(memory_space=pltpu.SMEM),
            pl.BlockSpec(memory_space=pl.ANY),
        ],
        out_specs=blk,
        out_shape=jax.ShapeDtypeStruct((_N, _N), jnp.float32),
        input_output_aliases={2: 0},
    )(dist, table, main)


def kernel(dist_par_0, par_table0):
    main = _sc_gather(dist_par_0, par_table0.reshape(-1))
    return _tail_fill(dist_par_0, par_table0, main)


# speedup vs baseline: 1.5984x; 1.0684x over previous
import functools

import jax
import jax.numpy as jnp
from jax import lax
from jax.experimental import pallas as pl
from jax.experimental.pallas import tpu as pltpu
from jax.experimental.pallas import tpu_sc as plsc

_N = 375
_ROWS_PER = 16
_FULL = _N // _ROWS_PER
_TAILW = _FULL
_TAIL_R0 = _N + 1 - _ROWS_PER
_COLS = list(range(0, _N - 16, 16)) + [_N - 16]
_NVALS = 14


def _make_sc_gather():
    mesh = plsc.VectorSubcoreMesh(core_axis_name="c", subcore_axis_name="s")

    @functools.partial(
        pl.kernel,
        mesh=mesh,
        out_type=jax.ShapeDtypeStruct((_N, _N), jnp.float32),
        scratch_types=[
            pltpu.VMEM((16,), jnp.float32),
            pltpu.VMEM((_ROWS_PER, _N), jnp.int32),
            pltpu.VMEM((_ROWS_PER, _N), jnp.float32),
            pltpu.SemaphoreType.DMA,
            pltpu.SemaphoreType.DMA,
        ],
    )
    def gather_kernel(idx_hbm, tab_hbm, out_hbm, tab_v, idx_v, val_v, s0, s1):
        wid = lax.axis_index("s") * 2 + lax.axis_index("c")

        dnums = lax.GatherDimensionNumbers(
            offset_dims=(), collapsed_slice_dims=(0,), start_index_map=(0,)
        )

        @pl.when(wid <= _TAILW)
        def _():
            r0 = pl.multiple_of(
                jnp.where(wid == _TAILW, _TAIL_R0, wid * _ROWS_PER), 8
            )
            tab_cp = pltpu.async_copy(tab_hbm, tab_v.at[pl.ds(0, _NVALS)], s0)
            idx_cp = pltpu.async_copy(
                idx_hbm.at[pl.ds(r0, _ROWS_PER)], idx_v, s1
            )
            tab_cp.wait()
            tab_vec = tab_v[...]
            idx_cp.wait()
            for r in range(_ROWS_PER):
                for c in _COLS:
                    idx16 = idx_v[r, pl.ds(c, 16)]
                    val_v[r, pl.ds(c, 16)] = lax.gather(
                        tab_vec,
                        idx16[:, None],
                        dnums,
                        slice_sizes=(1,),
                        mode=lax.GatherScatterMode.PROMISE_IN_BOUNDS,
                    )
            pltpu.sync_copy(val_v, out_hbm.at[pl.ds(r0, _ROWS_PER)])

    return gather_kernel


_sc_gather = _make_sc_gather()


def kernel(dist_par_0, par_table0):
    return _sc_gather(dist_par_0, par_table0.reshape(-1))


# speedup vs baseline: 1.6229x; 1.0153x over previous
import functools

import jax
import jax.numpy as jnp
from jax import lax
from jax.experimental import pallas as pl
from jax.experimental.pallas import tpu as pltpu
from jax.experimental.pallas import tpu_sc as plsc

_N = 375
_ROWS_PER = 16
_FULL = _N // _ROWS_PER
_TAILW = _FULL
_TAIL_R0 = _N + 1 - _ROWS_PER
_COLS = list(range(0, _N - 16, 16)) + [_N - 16]
_NVALS = 14


def _make_sc_gather():
    mesh = plsc.VectorSubcoreMesh(core_axis_name="c", subcore_axis_name="s")

    @functools.partial(
        pl.kernel,
        mesh=mesh,
        out_type=jax.ShapeDtypeStruct((_N, _N), jnp.float32),
        scratch_types=[
            pltpu.VMEM((16,), jnp.float32),
            pltpu.VMEM((_ROWS_PER, _N), jnp.int32),
            pltpu.VMEM((_ROWS_PER, _N), jnp.float32),
            pltpu.SemaphoreType.DMA,
            pltpu.SemaphoreType.DMA,
            pltpu.SemaphoreType.DMA,
            pltpu.SemaphoreType.DMA,
            pltpu.SemaphoreType.DMA,
        ],
    )
    def gather_kernel(idx_hbm, tab_hbm, out_hbm, tab_v, idx_v, val_v,
                      s0, s1, s2, s3, s4):
        wid = lax.axis_index("s") * 2 + lax.axis_index("c")

        dnums = lax.GatherDimensionNumbers(
            offset_dims=(), collapsed_slice_dims=(0,), start_index_map=(0,)
        )

        @pl.when(wid <= _TAILW)
        def _():
            r0 = pl.multiple_of(
                jnp.where(wid == _TAILW, _TAIL_R0, wid * _ROWS_PER), 8
            )
            half = _ROWS_PER // 2
            tab_cp = pltpu.async_copy(tab_hbm, tab_v.at[pl.ds(0, _NVALS)], s0)
            cp_a = pltpu.async_copy(
                idx_hbm.at[pl.ds(r0, half)], idx_v.at[pl.ds(0, half)], s1
            )
            cp_b = pltpu.async_copy(
                idx_hbm.at[pl.ds(r0 + half, half)],
                idx_v.at[pl.ds(half, half)],
                s2,
            )
            tab_cp.wait()
            tab_vec = tab_v[...]

            def lookup(r):
                for c in _COLS:
                    idx16 = idx_v[r, pl.ds(c, 16)]
                    val_v[r, pl.ds(c, 16)] = lax.gather(
                        tab_vec,
                        idx16[:, None],
                        dnums,
                        slice_sizes=(1,),
                        mode=lax.GatherScatterMode.PROMISE_IN_BOUNDS,
                    )

            cp_a.wait()
            for r in range(half):
                lookup(r)
            out_a = pltpu.async_copy(
                val_v.at[pl.ds(0, half)], out_hbm.at[pl.ds(r0, half)], s3
            )
            cp_b.wait()
            for r in range(half, _ROWS_PER):
                lookup(r)
            out_b = pltpu.async_copy(
                val_v.at[pl.ds(half, half)],
                out_hbm.at[pl.ds(r0 + half, half)],
                s4,
            )
            out_a.wait()
            out_b.wait()

    return gather_kernel


_sc_gather = _make_sc_gather()


def kernel(dist_par_0, par_table0):
    return _sc_gather(dist_par_0, par_table0.reshape(-1))


# speedup vs baseline: 1.6331x; 1.0063x over previous
import functools

import jax
import jax.numpy as jnp
from jax import lax
from jax.experimental import pallas as pl
from jax.experimental.pallas import tpu as pltpu
from jax.experimental.pallas import tpu_sc as plsc

_N = 375
_ROWS_PER = 16
_FULL = _N // _ROWS_PER
_TAILW = _FULL
_TAIL_R0 = _N + 1 - _ROWS_PER
_COLS = list(range(0, _N - 16, 16)) + [_N - 16]
_NVALS = 14


def _make_sc_gather():
    mesh = plsc.VectorSubcoreMesh(core_axis_name="c", subcore_axis_name="s")

    @functools.partial(
        pl.kernel,
        mesh=mesh,
        out_type=jax.ShapeDtypeStruct((_N, _N), jnp.float32),
        scratch_types=[
            pltpu.VMEM((16,), jnp.float32),
            pltpu.VMEM((_ROWS_PER, _N), jnp.int32),
            pltpu.VMEM((_ROWS_PER, _N), jnp.float32),
        ]
        + [pltpu.SemaphoreType.DMA] * 5,
    )
    def gather_kernel(idx_hbm, tab_hbm, out_hbm, tab_v, idx_v, val_v,
                      s0, *sems):
        wid = lax.axis_index("s") * 2 + lax.axis_index("c")

        dnums = lax.GatherDimensionNumbers(
            offset_dims=(), collapsed_slice_dims=(0,), start_index_map=(0,)
        )

        @pl.when(wid <= _TAILW)
        def _():
            r0 = pl.multiple_of(
                jnp.where(wid == _TAILW, _TAIL_R0, wid * _ROWS_PER), 8
            )
            q = _ROWS_PER // 2
            tab_cp = pltpu.async_copy(tab_hbm, tab_v.at[pl.ds(0, _NVALS)], s0)
            in_cps = [
                pltpu.async_copy(
                    idx_hbm.at[pl.ds(r0 + i * q, q)],
                    idx_v.at[pl.ds(i * q, q)],
                    sems[i],
                )
                for i in range(2)
            ]
            tab_cp.wait()
            tab_vec = tab_v[...]

            def lookup(r):
                for c in _COLS:
                    idx16 = idx_v[r, pl.ds(c, 16)]
                    val_v[r, pl.ds(c, 16)] = lax.gather(
                        tab_vec,
                        idx16[:, None],
                        dnums,
                        slice_sizes=(1,),
                        mode=lax.GatherScatterMode.PROMISE_IN_BOUNDS,
                    )

            out_cps = []
            for i in range(2):
                in_cps[i].wait()
                for r in range(i * q, (i + 1) * q):
                    lookup(r)
                out_cps.append(
                    pltpu.async_copy(
                        val_v.at[pl.ds(i * q, q)],
                        out_hbm.at[pl.ds(r0 + i * q, q)],
                        sems[2 + i],
                    )
                )
            for cp in out_cps:
                cp.wait()

    return gather_kernel


_sc_gather = _make_sc_gather()


def kernel(dist_par_0, par_table0):
    return _sc_gather(dist_par_0, par_table0.reshape(-1))
